# Initial kernel scaffold; baseline (speedup 1.0000x reference)
import jax
import jax.numpy as jnp
from jax.experimental import pallas as pl


def kernel(input_ids, token_embedding, position_embedding):
    raise NotImplementedError("write your pallas kernel here")


# speedup vs baseline: 2.2510x; 2.2510x over previous
import functools

import jax
import jax.numpy as jnp
from jax import lax
from jax.experimental import pallas as pl
from jax.experimental.pallas import tpu as pltpu
from jax.experimental.pallas import tpu_sc as plsc

BATCH = 4096
SEQ = 200
HIDDEN = 128
LANES = 16

NW = 32
ROWS = BATCH * SEQ
ROWS_PER_W = ROWS // NW
CHUNK = 128
NCHUNK = ROWS_PER_W // CHUNK
POS_EXT = SEQ + CHUNK


def _build():
    mesh = plsc.VectorSubcoreMesh(core_axis_name="c", subcore_axis_name="s")

    @functools.partial(
        pl.kernel,
        out_type=jax.ShapeDtypeStruct((ROWS, HIDDEN), jnp.float32),
        mesh=mesh,
        scratch_types=[
            pltpu.VMEM((NCHUNK, CHUNK), jnp.int32),
            pltpu.VMEM((POS_EXT, HIDDEN), jnp.float32),
            pltpu.VMEM((CHUNK, HIDDEN), jnp.float32),
            pltpu.SemaphoreType.DMA,
        ],
    )
    def emb_kernel(ids_hbm, tok_hbm, pos_hbm, out_hbm, idx_v, pos_v, buf_v, sem):
        wid = lax.axis_index("s") * 2 + lax.axis_index("c")
        base = wid * ROWS_PER_W

        pltpu.sync_copy(ids_hbm.at[wid], idx_v)
        pltpu.sync_copy(pos_hbm, pos_v)

        def chunk_body(j, carry):
            pltpu.async_copy(tok_hbm.at[idx_v.at[j]], buf_v, sem).wait()
            start = lax.rem(j * CHUNK, SEQ)

            def row_body(r, c):
                for g in range(HIDDEN // LANES):
                    vec = pos_v[start + r, pl.ds(g * LANES, LANES)]
                    plsc.addupdate(buf_v.at[r, pl.ds(g * LANES, LANES)], vec)
                return c

            lax.fori_loop(0, CHUNK, row_body, 0)
            pltpu.sync_copy(buf_v, out_hbm.at[pl.ds(base + j * CHUNK, CHUNK)])
            return carry

        lax.fori_loop(0, NCHUNK, chunk_body, 0)

    return emb_kernel


_emb = _build()


def kernel(input_ids, token_embedding, position_embedding):
    ids = input_ids.reshape(NW, NCHUNK, CHUNK).astype(jnp.int32)
    pos_ext = jnp.concatenate(
        [position_embedding, position_embedding[:CHUNK]], axis=0)
    out = _emb(ids, token_embedding, pos_ext)
    return out.reshape(BATCH, SEQ, HIDDEN)


# speedup vs baseline: 7.5344x; 3.3472x over previous
import functools

import jax
import jax.numpy as jnp
from jax import lax
from jax.experimental import pallas as pl
from jax.experimental.pallas import tpu as pltpu
from jax.experimental.pallas import tpu_sc as plsc

BATCH = 4096
SEQ = 200
HIDDEN = 128
LANES = 16

NW = 32
ROWS = BATCH * SEQ
ROWS_PER_W = ROWS // NW
SEQ_PER_W = ROWS_PER_W // SEQ
G0 = 128
G1 = SEQ - G0
NBUF = 2


def _build():
    mesh = plsc.VectorSubcoreMesh(core_axis_name="c", subcore_axis_name="s")

    @functools.partial(
        pl.kernel,
        out_type=jax.ShapeDtypeStruct((ROWS, HIDDEN), jnp.float32),
        mesh=mesh,
        scratch_types=[
            pltpu.VMEM((ROWS_PER_W,), jnp.int32),
            pltpu.VMEM((SEQ, HIDDEN), jnp.float32),
        ] + [pltpu.VMEM((SEQ, HIDDEN), jnp.float32) for _ in range(NBUF)]
          + [pltpu.SemaphoreType.DMA for _ in range(2 * NBUF)],
    )
    def emb_kernel(ids_hbm, tok_hbm, pos_hbm, out_hbm, idx_v, pos_v, *bs):
        bufs = bs[:NBUF]
        gsem = bs[NBUF:2 * NBUF]
        osem = bs[2 * NBUF:3 * NBUF]

        wid = lax.axis_index("s") * 2 + lax.axis_index("c")
        base = wid * ROWS_PER_W

        pltpu.sync_copy(ids_hbm.at[wid], idx_v)
        pltpu.sync_copy(pos_hbm, pos_v)

        def gather_parts(q, s):
            return (
                (tok_hbm.at[idx_v.at[pl.ds(q * SEQ, G0)]],
                 bufs[s].at[pl.ds(0, G0)]),
                (tok_hbm.at[idx_v.at[pl.ds(q * SEQ + G0, G1)]],
                 bufs[s].at[pl.ds(G0, G1)]),
            )

        def fire_gather(q, s):
            for src, dst in gather_parts(q, s):
                pltpu.async_copy(src, dst, gsem[s])

        def wait_gather(q, s):
            for src, dst in gather_parts(q, s):
                pltpu.make_async_copy(src, dst, gsem[s]).wait()

        for s in range(NBUF):
            fire_gather(s, s)

        @pl.loop(0, SEQ_PER_W, step=NBUF)
        def group(qb):
            for s in range(NBUF):
                q = qb + s
                wait_gather(q, s)

                @pl.loop(0, SEQ, unroll=8)
                def row(r):
                    for g in range(HIDDEN // LANES):
                        vec = pos_v[r, pl.ds(g * LANES, LANES)]
                        plsc.addupdate(bufs[s].at[r, pl.ds(g * LANES, LANES)],
                                       vec)

                dst = out_hbm.at[pl.ds(base + q * SEQ, SEQ)]
                pltpu.async_copy(bufs[s], dst, osem[s])

                @pl.when(q + NBUF < SEQ_PER_W)
                def _():
                    pltpu.make_async_copy(bufs[s], dst, osem[s]).wait()
                    fire_gather(q + NBUF, s)

        for s in range(NBUF):
            pltpu.make_async_copy(
                bufs[s], out_hbm.at[pl.ds(base, SEQ)], osem[s]).wait()

    return emb_kernel


_emb = _build()


def kernel(input_ids, token_embedding, position_embedding):
    ids = input_ids.reshape(NW, ROWS_PER_W).astype(jnp.int32)
    out = _emb(ids, token_embedding, position_embedding)
    return out.reshape(BATCH, SEQ, HIDDEN)
